# Initial kernel scaffold; baseline (speedup 1.0000x reference)
import jax
import jax.numpy as jnp
from jax.experimental import pallas as pl


def kernel(x):
    raise NotImplementedError("write your pallas kernel here")


# speedup vs baseline: 1.0432x; 1.0432x over previous
import jax
import jax.numpy as jnp
from jax.experimental import pallas as pl
from jax.experimental.pallas import tpu as pltpu

_B, _R, _C = 32, 4096, 1024
_NROWS = _B * _R
_BLOCK_ROWS = 512

_KS0 = 0
_KS1 = 42
_KS2 = 0x1BD11BDA ^ _KS0 ^ _KS1
_ROT0 = (13, 15, 26, 6)
_ROT1 = (17, 29, 16, 24)


def _rotl(x, r):
    return (x << jnp.uint32(r)) | (x >> jnp.uint32(32 - r))


def _threefry_bits(c1):
    ks0 = jnp.uint32(_KS0)
    ks1 = jnp.uint32(_KS1)
    ks2 = jnp.uint32(_KS2)
    x0 = ks0
    x1 = c1 + ks1

    def rounds(x0, x1, rots):
        for r in rots:
            x0 = x0 + x1
            x1 = _rotl(x1, r)
            x1 = x0 ^ x1
        return x0, x1

    x0, x1 = rounds(x0, x1, _ROT0)
    x0 = x0 + ks1
    x1 = x1 + (ks2 + jnp.uint32(1))
    x0, x1 = rounds(x0, x1, _ROT1)
    x0 = x0 + ks2
    x1 = x1 + (ks0 + jnp.uint32(2))
    x0, x1 = rounds(x0, x1, _ROT0)
    x0 = x0 + ks0
    x1 = x1 + (ks1 + jnp.uint32(3))
    x0, x1 = rounds(x0, x1, _ROT1)
    x0 = x0 + ks1
    x1 = x1 + (ks2 + jnp.uint32(4))
    x0, x1 = rounds(x0, x1, _ROT0)
    x0 = x0 + ks2
    x1 = x1 + (ks0 + jnp.uint32(5))
    return x0 ^ x1


def _body(x_ref, o_ref):
    g = pl.program_id(0)
    row0 = g.astype(jnp.uint32) * jnp.uint32(_BLOCK_ROWS)
    ri = jax.lax.broadcasted_iota(jnp.uint32, (_BLOCK_ROWS, _C), 0)
    ci = jax.lax.broadcasted_iota(jnp.uint32, (_BLOCK_ROWS, _C), 1)
    idx = ((row0 + ri) << jnp.uint32(10)) + ci
    bits = _threefry_bits(idx)
    u = pltpu.bitcast((bits >> jnp.uint32(9)) | jnp.uint32(0x3F800000),
                      jnp.float32) - jnp.float32(1.0)
    o_ref[...] = x_ref[...] + u


def kernel(x):
    x2 = x.reshape(_NROWS, _C)
    out = pl.pallas_call(
        _body,
        grid=(_NROWS // _BLOCK_ROWS,),
        in_specs=[pl.BlockSpec((_BLOCK_ROWS, _C), lambda g: (g, 0))],
        out_specs=pl.BlockSpec((_BLOCK_ROWS, _C), lambda g: (g, 0)),
        out_shape=jax.ShapeDtypeStruct((_NROWS, _C), jnp.float32),
        compiler_params=pltpu.CompilerParams(
            dimension_semantics=("parallel",)),
    )(x2)
    return out.reshape(_B, _R, _C)


# speedup vs baseline: 1.0559x; 1.0122x over previous
import jax
import jax.numpy as jnp
from jax.experimental import pallas as pl
from jax.experimental.pallas import tpu as pltpu

_B, _R, _C = 32, 4096, 1024
_NROWS = _B * _R
_BLOCK_ROWS = 512
_BLOCK = _BLOCK_ROWS * _C

_KS0 = 0
_KS1 = 42
_KS2 = 0x1BD11BDA ^ _KS0 ^ _KS1
_ROT0 = (13, 15, 26, 6)
_ROT1 = (17, 29, 16, 24)


def _rotl(x, r):
    return (x << jnp.uint32(r)) | (x >> jnp.uint32(32 - r))


def _threefry_bits(x1):
    ks0 = jnp.uint32(_KS0)
    ks1 = jnp.uint32(_KS1)
    ks2 = jnp.uint32(_KS2)
    x0 = ks0

    def rounds(x0, x1, rots):
        for r in rots:
            x0 = x0 + x1
            x1 = _rotl(x1, r)
            x1 = x0 ^ x1
        return x0, x1

    x0, x1 = rounds(x0, x1, _ROT0)
    x0 = x0 + ks1
    x1 = x1 + (ks2 + jnp.uint32(1))
    x0, x1 = rounds(x0, x1, _ROT1)
    x0 = x0 + ks2
    x1 = x1 + (ks0 + jnp.uint32(2))
    x0, x1 = rounds(x0, x1, _ROT0)
    x0 = x0 + ks0
    x1 = x1 + (ks1 + jnp.uint32(3))
    x0, x1 = rounds(x0, x1, _ROT1)
    x0 = x0 + ks1
    x1 = x1 + (ks2 + jnp.uint32(4))
    x0, x1 = rounds(x0, x1, _ROT0)
    x0 = x0 + ks2
    x1 = x1 + (ks0 + jnp.uint32(5))
    return x0 ^ x1


def _body(lin_ref, x_ref, o_ref):
    g = pl.program_id(0)
    off = g.astype(jnp.uint32) * jnp.uint32(_BLOCK)
    x1 = lin_ref[...] + off
    bits = _threefry_bits(x1)
    u = pltpu.bitcast((bits >> jnp.uint32(9)) | jnp.uint32(0x3F800000),
                      jnp.float32) - jnp.float32(1.0)
    o_ref[...] = x_ref[...] + u


def kernel(x):
    x2 = x.reshape(_NROWS, _C)
    lin = (jnp.arange(_BLOCK, dtype=jnp.uint32) + jnp.uint32(_KS1)).reshape(
        _BLOCK_ROWS, _C)
    out = pl.pallas_call(
        _body,
        grid=(_NROWS // _BLOCK_ROWS,),
        in_specs=[
            pl.BlockSpec((_BLOCK_ROWS, _C), lambda g: (0, 0)),
            pl.BlockSpec((_BLOCK_ROWS, _C), lambda g: (g, 0)),
        ],
        out_specs=pl.BlockSpec((_BLOCK_ROWS, _C), lambda g: (g, 0)),
        out_shape=jax.ShapeDtypeStruct((_NROWS, _C), jnp.float32),
        compiler_params=pltpu.CompilerParams(
            dimension_semantics=("arbitrary",)),
    )(lin, x2)
    return out.reshape(_B, _R, _C)


# speedup vs baseline: 1.0578x; 1.0017x over previous
import jax
import jax.numpy as jnp
from jax.experimental import pallas as pl
from jax.experimental.pallas import tpu as pltpu

_B, _R, _C = 32, 4096, 1024
_NROWS = _B * _R
_BLOCK_ROWS = 1024
_BLOCK = _BLOCK_ROWS * _C

_KS0 = 0
_KS1 = 42
_KS2 = 0x1BD11BDA ^ _KS0 ^ _KS1
_ROT0 = (13, 15, 26, 6)
_ROT1 = (17, 29, 16, 24)


def _rotl(x, r):
    return (x << jnp.uint32(r)) | (x >> jnp.uint32(32 - r))


def _threefry_bits(x1):
    ks0 = jnp.uint32(_KS0)
    ks1 = jnp.uint32(_KS1)
    ks2 = jnp.uint32(_KS2)
    x0 = ks0

    def rounds(x0, x1, rots):
        for r in rots:
            x0 = x0 + x1
            x1 = _rotl(x1, r)
            x1 = x0 ^ x1
        return x0, x1

    x0, x1 = rounds(x0, x1, _ROT0)
    x0 = x0 + ks1
    x1 = x1 + (ks2 + jnp.uint32(1))
    x0, x1 = rounds(x0, x1, _ROT1)
    x0 = x0 + ks2
    x1 = x1 + (ks0 + jnp.uint32(2))
    x0, x1 = rounds(x0, x1, _ROT0)
    x0 = x0 + ks0
    x1 = x1 + (ks1 + jnp.uint32(3))
    x0, x1 = rounds(x0, x1, _ROT1)
    x0 = x0 + ks1
    x1 = x1 + (ks2 + jnp.uint32(4))
    x0, x1 = rounds(x0, x1, _ROT0)
    x0 = x0 + ks2
    x1 = x1 + (ks0 + jnp.uint32(5))
    return x0 ^ x1


def _body(lin_ref, x_ref, o_ref):
    g = pl.program_id(0)
    off = g.astype(jnp.uint32) * jnp.uint32(_BLOCK)
    x1 = lin_ref[...] + off
    bits = _threefry_bits(x1)
    u = pltpu.bitcast((bits >> jnp.uint32(9)) | jnp.uint32(0x3F800000),
                      jnp.float32) - jnp.float32(1.0)
    o_ref[...] = x_ref[...] + u


def kernel(x):
    x2 = x.reshape(_NROWS, _C)
    lin = (jnp.arange(_BLOCK, dtype=jnp.uint32) + jnp.uint32(_KS1)).reshape(
        _BLOCK_ROWS, _C)
    out = pl.pallas_call(
        _body,
        grid=(_NROWS // _BLOCK_ROWS,),
        in_specs=[
            pl.BlockSpec((_BLOCK_ROWS, _C), lambda g: (0, 0)),
            pl.BlockSpec((_BLOCK_ROWS, _C), lambda g: (g, 0)),
        ],
        out_specs=pl.BlockSpec((_BLOCK_ROWS, _C), lambda g: (g, 0)),
        out_shape=jax.ShapeDtypeStruct((_NROWS, _C), jnp.float32),
        compiler_params=pltpu.CompilerParams(
            dimension_semantics=("arbitrary",),
            vmem_limit_bytes=50 * 1024 * 1024),
    )(lin, x2)
    return out.reshape(_B, _R, _C)
